# Initial kernel scaffold; baseline (speedup 1.0000x reference)
import jax
import jax.numpy as jnp
from jax.experimental import pallas as pl


def kernel(x, edge_index, surf_filter, Wf1, bf1, Ws1, bs1, Wf2, bf2, Ws2, bs2, Wlin, blin):
    raise NotImplementedError("write your pallas kernel here")


# speedup vs baseline: 1.9369x; 1.9369x over previous
import functools

import jax
import jax.numpy as jnp
from jax import lax
from jax.experimental import pallas as pl
from jax.experimental.pallas import tpu as pltpu, tpu_sc as plsc

N_NODES = 10000
N_EDGES = 320000
CH = 128
HC = CH // 2

NC = 2
NS = 16
EPS = N_EDGES // NS
K = 80
NCHUNK = EPS // K
N_PAD = 10240
ROWS_PER_TILE = N_PAD // NS
CP = K
NCP = ROWS_PER_TILE // CP
SB = 16


_RB = 1000


def _tables_body(x_ref, w_ref, b_ref, out_ref):
    p = jnp.dot(x_ref[...], w_ref[...], preferred_element_type=jnp.float32) + b_ref[...]
    for t in range(4):
        out_ref[t] = p[:, t * CH:(t + 1) * CH]


def _node_tables(h, w_all, b_all):
    return pl.pallas_call(
        _tables_body,
        grid=(N_NODES // _RB,),
        in_specs=[
            pl.BlockSpec((_RB, CH), lambda i: (i, 0)),
            pl.BlockSpec((CH, 4 * CH), lambda i: (0, 0)),
            pl.BlockSpec((1, 4 * CH), lambda i: (0, 0)),
        ],
        out_specs=pl.BlockSpec((4, _RB, CH), lambda i: (0, i, 0)),
        out_shape=jax.ShapeDtypeStruct((4, N_NODES, CH), jnp.float32),
    )(h, w_all, b_all)


def _combine_tables_body(agg_ref, x_ref, w_ref, b_ref, h_ref, out_ref):
    hb = agg_ref[0] + agg_ref[1] + x_ref[...]
    h_ref[...] = hb
    p = jnp.dot(hb, w_ref[...], preferred_element_type=jnp.float32) + b_ref[...]
    for t in range(4):
        out_ref[t] = p[:, t * CH:(t + 1) * CH]


def _combine_and_tables(agg, x, w_all, b_all):
    return pl.pallas_call(
        _combine_tables_body,
        grid=(N_NODES // _RB,),
        in_specs=[
            pl.BlockSpec((NC, _RB, CH), lambda i: (0, i, 0)),
            pl.BlockSpec((_RB, CH), lambda i: (i, 0)),
            pl.BlockSpec((CH, 4 * CH), lambda i: (0, 0)),
            pl.BlockSpec((1, 4 * CH), lambda i: (0, 0)),
        ],
        out_specs=[
            pl.BlockSpec((_RB, CH), lambda i: (i, 0)),
            pl.BlockSpec((4, _RB, CH), lambda i: (0, i, 0)),
        ],
        out_shape=[
            jax.ShapeDtypeStruct((N_NODES, CH), jnp.float32),
            jax.ShapeDtypeStruct((4, N_NODES, CH), jnp.float32),
        ],
    )(agg, x, w_all, b_all)


def _readout_body(agg_ref, h1_ref, surf_ref, wl_ref, bl_ref, y_ref):
    h2 = agg_ref[0, :N_NODES] + agg_ref[1, :N_NODES] + h1_ref[...]
    surf = surf_ref[...]
    sv = jnp.sum(surf * h2, axis=0)
    num = jnp.sum(sv * wl_ref[...][:, 0]) + N_NODES * bl_ref[0, 0]
    y_ref[...] = jnp.reshape(num / jnp.sum(surf), (1, 1))


def _readout(agg, h1, surf2d, wlin, blin2d):
    return pl.pallas_call(
        _readout_body,
        out_shape=jax.ShapeDtypeStruct((1, 1), jnp.float32),
    )(agg, h1, surf2d, wlin, blin2d)


_P0 = 0.99999014
_P1 = 0.57257779
_P2 = 0.01698017
_Q1 = 1.07241614
_Q2 = 0.22081657


def _gated_msg(u, v):
    a = jnp.exp(-u)
    t = jnp.exp(-jnp.abs(v))
    p = t * (_P0 + t * (_P1 + t * _P2))
    q = 1.0 + t * (_Q1 + t * _Q2)
    num = jnp.maximum(v, 0.0) * q + p
    den = q * (1.0 + a)
    return num / den


def _edge_kernel(tall_hbm, dsti_hbm, srci_hbm, agg_hbm,
                 dstA, dgtA, sgtA, dstB, dgtB, sgtB,
                 dbufA, sbufA, dbufB, sbufB, m0, m1, agg_sh,
                 semAd, semAs, semBd, semBs, semS0, semS1):
    cid = lax.axis_index("c")
    sid = lax.axis_index("s")
    ebase = sid * EPS
    doff = cid * N_NODES
    soff = 2 * N_NODES + cid * N_NODES
    moff = cid * HC

    def zrow(r, carry):
        for g in range(CH // 16):
            dbufA[r, pl.ds(g * 16, 16)] = jnp.zeros((16,), jnp.float32)
        return carry
    lax.fori_loop(0, K, zrow, 0)
    for j in range(NCP):
        pltpu.sync_copy(dbufA, agg_sh.at[pl.ds(sid * ROWS_PER_TILE + j * CP, CP)])

    def zmrow(r, carry):
        for g in range(CH // 16):
            m0[r, pl.ds(g * 16, 16)] = jnp.zeros((16,), jnp.float32)
            m1[r, pl.ds(g * 16, 16)] = jnp.zeros((16,), jnp.float32)
        return carry
    lax.fori_loop(0, SB, zmrow, 0)
    plsc.subcore_barrier()

    pad_idx = lax.iota(jnp.int32, 16) + N_NODES
    pltpu.async_copy(m0, agg_sh.at[pad_idx], semS0, add=True)
    pltpu.async_copy(m1, agg_sh.at[pad_idx], semS1, add=True)

    def load_idx(dst_v, dgt_v, sgt_v, ci):
        b = ebase + ci * K
        pltpu.sync_copy(dsti_hbm.at[pl.ds(b, K)], dst_v)
        pltpu.sync_copy(srci_hbm.at[pl.ds(b, K)], sgt_v)
        for j in range(K // 16):
            dgt_v[pl.ds(j * 16, 16)] = dst_v[pl.ds(j * 16, 16)] + doff
            sgt_v[pl.ds(j * 16, 16)] = sgt_v[pl.ds(j * 16, 16)] + soff

    def start_gather(dgt_v, sgt_v, dbuf, sbuf, sd, ss):
        pltpu.async_copy(tall_hbm.at[dgt_v], dbuf, sd)
        pltpu.async_copy(tall_hbm.at[sgt_v], sbuf, ss)

    def wait_gather(dgt_v, sgt_v, dbuf, sbuf, sd, ss):
        pltpu.make_async_copy(tall_hbm.at[dgt_v], dbuf, sd).wait()
        pltpu.make_async_copy(tall_hbm.at[sgt_v], sbuf, ss).wait()

    def compute_chunk(dbuf, sbuf, dst_v):
        for sb in range(K // SB):
            mb, sem = (m0, semS0) if sb % 2 == 0 else (m1, semS1)
            idx = dst_v[pl.ds(sb * SB, 16)]
            pltpu.make_async_copy(mb, agg_sh.at[idx], sem).wait()

            @plsc.parallel_loop(0, SB, unroll=4)
            def eb(e):
                r = sb * SB + e
                for g in range(HC // 16):
                    u = dbuf[r, pl.ds(g * 16, 16)] + sbuf[r, pl.ds(g * 16, 16)]
                    v = dbuf[r, pl.ds(HC + g * 16, 16)] + sbuf[r, pl.ds(HC + g * 16, 16)]
                    mb[e, pl.ds(moff + g * 16, 16)] = _gated_msg(u, v)
            pltpu.async_copy(mb, agg_sh.at[idx], semS0 if sb % 2 == 0 else semS1, add=True)

    load_idx(dstA, dgtA, sgtA, 0)
    start_gather(dgtA, sgtA, dbufA, sbufA, semAd, semAs)

    def body(t, carry):
        load_idx(dstB, dgtB, sgtB, 2 * t + 1)
        start_gather(dgtB, sgtB, dbufB, sbufB, semBd, semBs)
        wait_gather(dgtA, sgtA, dbufA, sbufA, semAd, semAs)
        compute_chunk(dbufA, sbufA, dstA)
        ca = jnp.minimum(2 * t + 2, NCHUNK - 2)
        load_idx(dstA, dgtA, sgtA, ca)
        start_gather(dgtA, sgtA, dbufA, sbufA, semAd, semAs)
        wait_gather(dgtB, sgtB, dbufB, sbufB, semBd, semBs)
        compute_chunk(dbufB, sbufB, dstB)
        return carry
    lax.fori_loop(0, NCHUNK // 2, body, 0)

    wait_gather(dgtA, sgtA, dbufA, sbufA, semAd, semAs)
    pltpu.make_async_copy(m0, agg_sh.at[pad_idx], semS0).wait()
    pltpu.make_async_copy(m1, agg_sh.at[pad_idx], semS1).wait()
    plsc.subcore_barrier()

    for j in range(NCP):
        r0 = sid * ROWS_PER_TILE + j * CP
        pltpu.sync_copy(agg_sh.at[pl.ds(r0, CP)], dbufA)
        pltpu.sync_copy(dbufA, agg_hbm.at[cid].at[pl.ds(r0, CP)])


def _edge_stage(table, dst_idx, src_idx):
    mesh = plsc.VectorSubcoreMesh(core_axis_name="c", subcore_axis_name="s")
    f = functools.partial(
        pl.kernel,
        out_type=jax.ShapeDtypeStruct((NC, N_PAD, CH), jnp.float32),
        mesh=mesh,
        scratch_types=[
            pltpu.VMEM((K,), jnp.int32),
            pltpu.VMEM((K,), jnp.int32),
            pltpu.VMEM((K,), jnp.int32),
            pltpu.VMEM((K,), jnp.int32),
            pltpu.VMEM((K,), jnp.int32),
            pltpu.VMEM((K,), jnp.int32),
            pltpu.VMEM((K, CH), jnp.float32),
            pltpu.VMEM((K, CH), jnp.float32),
            pltpu.VMEM((K, CH), jnp.float32),
            pltpu.VMEM((K, CH), jnp.float32),
            pltpu.VMEM((SB, CH), jnp.float32),
            pltpu.VMEM((SB, CH), jnp.float32),
            pltpu.VMEM_SHARED((N_PAD, CH), jnp.float32),
            pltpu.SemaphoreType.DMA,
            pltpu.SemaphoreType.DMA,
            pltpu.SemaphoreType.DMA,
            pltpu.SemaphoreType.DMA,
            pltpu.SemaphoreType.DMA,
            pltpu.SemaphoreType.DMA,
        ],
    )(_edge_kernel)
    return f(table, dst_idx, src_idx)


def _pack_weights(Wf, bf, Ws, bs):
    blocks = []
    bias = []
    for half in (slice(0, CH), slice(CH, 2 * CH)):
        for c in range(NC):
            cs = slice(c * HC, (c + 1) * HC)
            blocks += [Wf[half, cs], Ws[half, cs]]
    bias = jnp.concatenate([bf[0:HC], bs[0:HC], bf[HC:CH], bs[HC:CH],
                            jnp.zeros((2 * CH,), jnp.float32)])
    return jnp.concatenate(blocks, axis=1), bias.reshape(1, 4 * CH)


def kernel(x, edge_index, surf_filter, Wf1, bf1, Ws1, bs1, Wf2, bf2, Ws2, bs2, Wlin, blin):
    ei = edge_index.astype(jnp.int32)
    src = ei[0]
    dst = ei[1]

    w1, b1 = _pack_weights(Wf1, bf1, Ws1, bs1)
    w2, b2 = _pack_weights(Wf2, bf2, Ws2, bs2)

    t1 = _node_tables(x, w1, b1).reshape(4 * N_NODES, CH)
    agg1 = _edge_stage(t1, dst, src)
    h1, t2 = _combine_and_tables(agg1[:, :N_NODES, :], x, w2, b2)
    agg2 = _edge_stage(t2.reshape(4 * N_NODES, CH), dst, src)
    y = _readout(agg2, h1, surf_filter.reshape(N_NODES, 1), Wlin, blin.reshape(1, 1))
    return y[0, 0]


# speedup vs baseline: 2.6355x; 1.3607x over previous
import functools

import jax
import jax.numpy as jnp
from jax import lax
from jax.experimental import pallas as pl
from jax.experimental.pallas import tpu as pltpu, tpu_sc as plsc

N_NODES = 10000
N_EDGES = 320000
CH = 128
HC = CH // 2

NC = 2
NS = 16
EPS = N_EDGES // NS
K = 80
NCHUNK = EPS // K
N_PAD = 10240
ROWS_PER_TILE = N_PAD // NS
CP = K
NCP = ROWS_PER_TILE // CP
SB = 16


_RB = 1000


def _tables_body(x_ref, w_ref, b_ref, out_ref):
    p = jnp.dot(x_ref[...], w_ref[...], preferred_element_type=jnp.float32) + b_ref[...]
    for t in range(4):
        out_ref[t] = p[:, t * CH:(t + 1) * CH]


def _node_tables(h, w_all, b_all):
    return pl.pallas_call(
        _tables_body,
        grid=(N_NODES // _RB,),
        in_specs=[
            pl.BlockSpec((_RB, CH), lambda i: (i, 0)),
            pl.BlockSpec((CH, 4 * CH), lambda i: (0, 0)),
            pl.BlockSpec((1, 4 * CH), lambda i: (0, 0)),
        ],
        out_specs=pl.BlockSpec((4, _RB, CH), lambda i: (0, i, 0)),
        out_shape=jax.ShapeDtypeStruct((4, N_NODES, CH), jnp.float32),
    )(h, w_all, b_all)


def _combine_tables_body(agg_ref, x_ref, w_ref, b_ref, h_ref, out_ref):
    hb = agg_ref[0] + agg_ref[1] + x_ref[...]
    h_ref[...] = hb
    p = jnp.dot(hb, w_ref[...], preferred_element_type=jnp.float32) + b_ref[...]
    for t in range(4):
        out_ref[t] = p[:, t * CH:(t + 1) * CH]


def _combine_and_tables(agg, x, w_all, b_all):
    return pl.pallas_call(
        _combine_tables_body,
        grid=(N_NODES // _RB,),
        in_specs=[
            pl.BlockSpec((NC, _RB, CH), lambda i: (0, i, 0)),
            pl.BlockSpec((_RB, CH), lambda i: (i, 0)),
            pl.BlockSpec((CH, 4 * CH), lambda i: (0, 0)),
            pl.BlockSpec((1, 4 * CH), lambda i: (0, 0)),
        ],
        out_specs=[
            pl.BlockSpec((_RB, CH), lambda i: (i, 0)),
            pl.BlockSpec((4, _RB, CH), lambda i: (0, i, 0)),
        ],
        out_shape=[
            jax.ShapeDtypeStruct((N_NODES, CH), jnp.float32),
            jax.ShapeDtypeStruct((4, N_NODES, CH), jnp.float32),
        ],
    )(agg, x, w_all, b_all)


def _readout_body(agg_ref, h1_ref, surf_ref, wl_ref, bl_ref, y_ref):
    h2 = agg_ref[0, :N_NODES] + agg_ref[1, :N_NODES] + h1_ref[...]
    surf = surf_ref[...]
    sv = jnp.sum(surf * h2, axis=0)
    num = jnp.sum(sv * wl_ref[...][:, 0]) + N_NODES * bl_ref[0, 0]
    y_ref[...] = jnp.reshape(num / jnp.sum(surf), (1, 1))


def _readout(agg, h1, surf2d, wlin, blin2d):
    return pl.pallas_call(
        _readout_body,
        out_shape=jax.ShapeDtypeStruct((1, 1), jnp.float32),
    )(agg, h1, surf2d, wlin, blin2d)


_P0 = 0.99999014
_P1 = 0.57257779
_P2 = 0.01698017
_Q1 = 1.07241614
_Q2 = 0.22081657


def _gated_msg(u, v):
    a = jnp.exp(-u)
    t = jnp.exp(-jnp.abs(v))
    p = t * (_P0 + t * (_P1 + t * _P2))
    q = 1.0 + t * (_Q1 + t * _Q2)
    num = jnp.maximum(v, 0.0) * q + p
    den = q * (1.0 + a)
    return num / den


def _edge_kernel(tall_hbm, dsti_hbm, srci_hbm, agg_hbm,
                 dstA, dgtA, sgtA, dstB, dgtB, sgtB, pd0, ps0, pd1, ps1,
                 dbufA, sbufA, dbufB, sbufB, m0, m1, agg_sh,
                 semAd, semAs, semBd, semBs, semS0, semS1, semIA, semIB):
    cid = lax.axis_index("c")
    sid = lax.axis_index("s")
    ebase = sid * EPS
    doff = cid * N_NODES
    soff = 2 * N_NODES + cid * N_NODES
    moff = cid * HC

    def zrow(r, carry):
        for g in range(CH // 16):
            dbufA[r, pl.ds(g * 16, 16)] = jnp.zeros((16,), jnp.float32)
        return carry
    lax.fori_loop(0, K, zrow, 0)
    for j in range(NCP):
        pltpu.sync_copy(dbufA, agg_sh.at[pl.ds(sid * ROWS_PER_TILE + j * CP, CP)])

    def zmrow(r, carry):
        for g in range(CH // 16):
            m0[r, pl.ds(g * 16, 16)] = jnp.zeros((16,), jnp.float32)
            m1[r, pl.ds(g * 16, 16)] = jnp.zeros((16,), jnp.float32)
        return carry
    lax.fori_loop(0, SB, zmrow, 0)
    plsc.subcore_barrier()

    pad_idx = lax.iota(jnp.int32, 16) + N_NODES
    pltpu.async_copy(m0, agg_sh.at[pad_idx], semS0, add=True)
    pltpu.async_copy(m1, agg_sh.at[pad_idx], semS1, add=True)

    def start_idx(pd, ps, ci, sem):
        b = ebase + ci * K
        pltpu.async_copy(dsti_hbm.at[pl.ds(b, K)], pd, sem)
        pltpu.async_copy(srci_hbm.at[pl.ds(b, K)], ps, sem)

    def finish_idx(pd, ps, dst_v, dgt_v, sgt_v, ci, sem):
        b = ebase + ci * K
        pltpu.make_async_copy(dsti_hbm.at[pl.ds(b, K)], pd, sem).wait()
        pltpu.make_async_copy(srci_hbm.at[pl.ds(b, K)], ps, sem).wait()
        for j in range(K // 16):
            d = pd[pl.ds(j * 16, 16)]
            dst_v[pl.ds(j * 16, 16)] = d
            dgt_v[pl.ds(j * 16, 16)] = d + doff
            sgt_v[pl.ds(j * 16, 16)] = ps[pl.ds(j * 16, 16)] + soff

    def start_gather(dgt_v, sgt_v, dbuf, sbuf, sd, ss):
        pltpu.async_copy(tall_hbm.at[dgt_v], dbuf, sd)
        pltpu.async_copy(tall_hbm.at[sgt_v], sbuf, ss)

    def wait_gather(dgt_v, sgt_v, dbuf, sbuf, sd, ss):
        pltpu.make_async_copy(tall_hbm.at[dgt_v], dbuf, sd).wait()
        pltpu.make_async_copy(tall_hbm.at[sgt_v], sbuf, ss).wait()

    def compute_chunk(dbuf, sbuf, dst_v):
        for sb in range(K // SB):
            mb, sem = (m0, semS0) if sb % 2 == 0 else (m1, semS1)
            idx = dst_v[pl.ds(sb * SB, 16)]
            pltpu.make_async_copy(mb, agg_sh.at[idx], sem).wait()

            @plsc.parallel_loop(0, SB, unroll=4)
            def eb(e):
                r = sb * SB + e
                for g in range(HC // 16):
                    u = dbuf[r, pl.ds(g * 16, 16)] + sbuf[r, pl.ds(g * 16, 16)]
                    v = dbuf[r, pl.ds(HC + g * 16, 16)] + sbuf[r, pl.ds(HC + g * 16, 16)]
                    mb[e, pl.ds(moff + g * 16, 16)] = _gated_msg(u, v)
            pltpu.async_copy(mb, agg_sh.at[idx], semS0 if sb % 2 == 0 else semS1, add=True)

    start_idx(pd0, ps0, 0, semIA)
    finish_idx(pd0, ps0, dstA, dgtA, sgtA, 0, semIA)
    start_gather(dgtA, sgtA, dbufA, sbufA, semAd, semAs)
    start_idx(pd1, ps1, 1, semIB)

    def body(t, carry):
        finish_idx(pd1, ps1, dstB, dgtB, sgtB, 2 * t + 1, semIB)
        start_gather(dgtB, sgtB, dbufB, sbufB, semBd, semBs)
        ca = jnp.minimum(2 * t + 2, NCHUNK - 1)
        start_idx(pd0, ps0, ca, semIA)
        wait_gather(dgtA, sgtA, dbufA, sbufA, semAd, semAs)
        compute_chunk(dbufA, sbufA, dstA)
        finish_idx(pd0, ps0, dstA, dgtA, sgtA, ca, semIA)
        start_gather(dgtA, sgtA, dbufA, sbufA, semAd, semAs)
        cb = jnp.minimum(2 * t + 3, NCHUNK - 1)
        start_idx(pd1, ps1, cb, semIB)
        wait_gather(dgtB, sgtB, dbufB, sbufB, semBd, semBs)
        compute_chunk(dbufB, sbufB, dstB)
        return carry
    lax.fori_loop(0, NCHUNK // 2, body, 0)

    wait_gather(dgtA, sgtA, dbufA, sbufA, semAd, semAs)
    pltpu.make_async_copy(dsti_hbm.at[pl.ds(ebase + (NCHUNK - 1) * K, K)], pd1, semIB).wait()
    pltpu.make_async_copy(srci_hbm.at[pl.ds(ebase + (NCHUNK - 1) * K, K)], ps1, semIB).wait()
    pltpu.make_async_copy(m0, agg_sh.at[pad_idx], semS0).wait()
    pltpu.make_async_copy(m1, agg_sh.at[pad_idx], semS1).wait()
    plsc.subcore_barrier()

    for j in range(NCP):
        r0 = sid * ROWS_PER_TILE + j * CP
        pltpu.sync_copy(agg_sh.at[pl.ds(r0, CP)], dbufA)
        pltpu.sync_copy(dbufA, agg_hbm.at[cid].at[pl.ds(r0, CP)])


def _edge_stage(table, dst_idx, src_idx):
    mesh = plsc.VectorSubcoreMesh(core_axis_name="c", subcore_axis_name="s")
    f = functools.partial(
        pl.kernel,
        out_type=jax.ShapeDtypeStruct((NC, N_PAD, CH), jnp.float32),
        mesh=mesh,
        scratch_types=[
            pltpu.VMEM((K,), jnp.int32),
            pltpu.VMEM((K,), jnp.int32),
            pltpu.VMEM((K,), jnp.int32),
            pltpu.VMEM((K,), jnp.int32),
            pltpu.VMEM((K,), jnp.int32),
            pltpu.VMEM((K,), jnp.int32),
            pltpu.VMEM((K,), jnp.int32),
            pltpu.VMEM((K,), jnp.int32),
            pltpu.VMEM((K,), jnp.int32),
            pltpu.VMEM((K,), jnp.int32),
            pltpu.VMEM((K, CH), jnp.float32),
            pltpu.VMEM((K, CH), jnp.float32),
            pltpu.VMEM((K, CH), jnp.float32),
            pltpu.VMEM((K, CH), jnp.float32),
            pltpu.VMEM((SB, CH), jnp.float32),
            pltpu.VMEM((SB, CH), jnp.float32),
            pltpu.VMEM_SHARED((N_PAD, CH), jnp.float32),
            pltpu.SemaphoreType.DMA,
            pltpu.SemaphoreType.DMA,
            pltpu.SemaphoreType.DMA,
            pltpu.SemaphoreType.DMA,
            pltpu.SemaphoreType.DMA,
            pltpu.SemaphoreType.DMA,
            pltpu.SemaphoreType.DMA,
            pltpu.SemaphoreType.DMA,
        ],
    )(_edge_kernel)
    return f(table, dst_idx, src_idx)


def _pack_weights(Wf, bf, Ws, bs):
    blocks = []
    bias = []
    for half in (slice(0, CH), slice(CH, 2 * CH)):
        for c in range(NC):
            cs = slice(c * HC, (c + 1) * HC)
            blocks += [Wf[half, cs], Ws[half, cs]]
    bias = jnp.concatenate([bf[0:HC], bs[0:HC], bf[HC:CH], bs[HC:CH],
                            jnp.zeros((2 * CH,), jnp.float32)])
    return jnp.concatenate(blocks, axis=1), bias.reshape(1, 4 * CH)


def kernel(x, edge_index, surf_filter, Wf1, bf1, Ws1, bs1, Wf2, bf2, Ws2, bs2, Wlin, blin):
    ei = edge_index.astype(jnp.int32)
    src = ei[0]
    dst = ei[1]

    w1, b1 = _pack_weights(Wf1, bf1, Ws1, bs1)
    w2, b2 = _pack_weights(Wf2, bf2, Ws2, bs2)

    t1 = _node_tables(x, w1, b1).reshape(4 * N_NODES, CH)
    agg1 = _edge_stage(t1, dst, src)
    h1, t2 = _combine_and_tables(agg1[:, :N_NODES, :], x, w2, b2)
    agg2 = _edge_stage(t2.reshape(4 * N_NODES, CH), dst, src)
    y = _readout(agg2, h1, surf_filter.reshape(N_NODES, 1), Wlin, blin.reshape(1, 1))
    return y[0, 0]
